# Initial kernel scaffold; baseline (speedup 1.0000x reference)
import jax
import jax.numpy as jnp
from jax.experimental import pallas as pl


def kernel(pred, gold):
    raise NotImplementedError("write your pallas kernel here")


# speedup vs baseline: 2.2783x; 2.2783x over previous
import math

import jax
import jax.numpy as jnp
from jax.experimental import pallas as pl
from jax.experimental.pallas import tpu as pltpu

_SMOOTHING = 0.1
_CONF = 1.0 - _SMOOTHING
_BLK_W = 2048


def _loss_kernel(gold_ref, pred_ref, out_ref, *, n_blk, blk_w, V, B):
    j = pl.program_id(0)
    s = _SMOOTHING / (V - 2)
    x = pred_ref[...]
    gold = gold_ref[...]
    cols = j * blk_w + jax.lax.broadcasted_iota(jnp.int32, x.shape, 1)
    valid = cols < V
    part_all = jnp.sum(jnp.where(valid, x, 0.0))
    pg_part = jnp.sum(jnp.where(cols == gold, x, 0.0))

    @pl.when(j == 0)
    def _init():
        k1 = (V - 2) * s * math.log(s) + _CONF * math.log(_CONF)
        p0 = x[:, 0:1]
        gz = gold == 0
        s0 = jnp.sum(p0)
        n0 = jnp.sum(gz.astype(jnp.float32))
        s00 = jnp.sum(jnp.where(gz, p0, 0.0))
        out_ref[0, 0] = (B * k1 + s * s0 + n0 * (s * math.log(s)) - s * s00)

    out_ref[0, 0] += (-s) * part_all + (s - _CONF) * pg_part


def kernel(pred, gold):
    B, V = pred.shape
    blk_w = _BLK_W
    n_blk = pl.cdiv(V, blk_w)
    gold2 = gold.reshape(B, 1)
    out = pl.pallas_call(
        lambda g, p, o: _loss_kernel(g, p, o, n_blk=n_blk, blk_w=blk_w, V=V, B=B),
        grid=(n_blk,),
        in_specs=[
            pl.BlockSpec((B, 1), lambda j: (0, 0)),
            pl.BlockSpec((B, blk_w), lambda j: (0, j)),
        ],
        out_specs=pl.BlockSpec(memory_space=pltpu.SMEM),
        out_shape=jax.ShapeDtypeStruct((1, 1), jnp.float32),
        compiler_params=pltpu.CompilerParams(
            dimension_semantics=("arbitrary",),
        ),
    )(gold2, pred)
    return out[0, 0]
